# Initial kernel scaffold; baseline (speedup 1.0000x reference)
import jax
import jax.numpy as jnp
from jax.experimental import pallas as pl


def kernel(feat, edge_index_rel0, edge_index_rel1, W1_0, b1_0, W1_1, b1_1, W2_0, b2_0, W2_1, b2_1):
    raise NotImplementedError("write your pallas kernel here")


# speedup vs baseline: 3.6649x; 3.6649x over previous
import jax
import jax.numpy as jnp
from jax import lax
from jax.experimental import pallas as pl
from jax.experimental.pallas import tpu as pltpu
from jax.experimental.pallas import tpu_sc as plsc

N = 10000
E = 160000
D_IN = 128
D_HID = 128
D_OUT = 64

N_TILES = 16
NPAD = 10240
RPT = NPAD // N_TILES
CHUNK = 128
NCHUNK = RPT // CHUNK

BLK = 128
EPT = 10240
ROWS_PT = EPT // BLK
KB = 4
GROUPS = ROWS_PT // KB
EPAD = EPT * N_TILES
EROWS = EPAD // BLK


def _sc_agg(D):
    mesh = plsc.VectorSubcoreMesh(core_axis_name="c", subcore_axis_name="s")
    outs = [jax.ShapeDtypeStruct((NPAD, D), jnp.float32),
            jax.ShapeDtypeStruct((NPAD, D), jnp.float32)]
    scratch = [
        pltpu.VMEM((BLK,), jnp.int32),
        pltpu.VMEM((BLK,), jnp.int32),
        pltpu.VMEM((BLK, D), jnp.float32),
        pltpu.VMEM_SHARED((NPAD, D), jnp.float32),
        pltpu.SemaphoreType.DMA,
    ]

    def body(t0, t1, src0, dst0, src1, dst1, zrow,
             out0, out1, srci, dsti, rows, acc, sem):
        s = lax.axis_index("s")
        c = lax.axis_index("c")

        pltpu.sync_copy(zrow, rows)
        for k in range(NCHUNK):
            pltpu.sync_copy(rows, acc.at[pl.ds(s * RPT + k * CHUNK, CHUNK)])
        plsc.subcore_barrier()

        def run(table, srch, dsth):
            @pl.loop(0, ROWS_PT)
            def step(r):
                row = s * ROWS_PT + r
                pltpu.sync_copy(srch.at[row], srci)
                pltpu.sync_copy(dsth.at[row], dsti)
                pltpu.async_copy(table.at[srci], rows, sem).wait()
                pltpu.sync_copy(rows, acc.at[dsti], add=True)

        pl.when(c == 0)(lambda: run(t0, src0, dst0))
        pl.when(c == 1)(lambda: run(t1, src1, dst1))
        plsc.subcore_barrier()

        def flush(outh):
            for k in range(NCHUNK):
                sl = pl.ds(s * RPT + k * CHUNK, CHUNK)
                pltpu.sync_copy(acc.at[sl], rows)
                pltpu.sync_copy(rows, outh.at[sl])

        pl.when(c == 0)(lambda: flush(out0))
        pl.when(c == 1)(lambda: flush(out1))

    return pl.kernel(body, out_type=tuple(outs), mesh=mesh,
                     scratch_types=tuple(scratch))


RB = 1000


def _tc_mid_body(a0r, a1r, c0r, c1r, w10r, b10r, w11r, b11r,
                 w20r, b20r, w21r, b21r, o0r):
    cc0 = c0r[...][:, 0:1]
    cc1 = c1r[...][:, 0:1]
    n0 = a0r[...] / jnp.maximum(cc0, 1.0)
    n1 = a1r[...] / jnp.maximum(cc1, 1.0)
    m0 = (cc0 > 0.0).astype(jnp.float32)
    m1 = (cc1 > 0.0).astype(jnp.float32)
    h = (jnp.dot(n0, w10r[...], preferred_element_type=jnp.float32)
         + m0 * b10r[...]
         + jnp.dot(n1, w11r[...], preferred_element_type=jnp.float32)
         + m1 * b11r[...])
    h = jnp.maximum(h, 0.0)
    o0r[:, 0:D_OUT] = jnp.dot(h, w20r[...], preferred_element_type=jnp.float32) + b20r[...]
    o0r[:, D_OUT:2 * D_OUT] = jnp.dot(h, w21r[...], preferred_element_type=jnp.float32) + b21r[...]


def _tc_mid(a0, a1, c0, c1, W1_0, b1_0, W1_1, b1_1, W2_0, b2_0, W2_1, b2_1):
    grid = (N // RB,)
    full = lambda shape: pl.BlockSpec(shape, lambda i: (0, 0))
    row = lambda shape: pl.BlockSpec(shape, lambda i: (i, 0))
    return pl.pallas_call(
        _tc_mid_body,
        grid=grid,
        in_specs=[row((RB, D_IN)), row((RB, D_IN)),
                  row((RB, 16)), row((RB, 16)),
                  full((D_IN, D_HID)), full((1, D_HID)),
                  full((D_IN, D_HID)), full((1, D_HID)),
                  full((D_HID, D_OUT)), full((1, D_OUT)),
                  full((D_HID, D_OUT)), full((1, D_OUT))],
        out_specs=row((RB, 2 * D_OUT)),
        out_shape=jax.ShapeDtypeStruct((N, 2 * D_OUT), jnp.float32),
    )(a0, a1, c0, c1,
      W1_0, b1_0.reshape(1, -1), W1_1, b1_1.reshape(1, -1),
      W2_0, b2_0.reshape(1, -1), W2_1, b2_1.reshape(1, -1))


def _tc_final_body(a0r, a1r, c0r, c1r, o):
    cc0 = c0r[...][:, 0:1]
    cc1 = c1r[...][:, 0:1]
    o[...] = (a0r[...] / jnp.maximum(cc0, 1.0)
              + a1r[...] / jnp.maximum(cc1, 1.0))


def _tc_final(a0, a1, c0, c1):
    grid = (N // RB,)
    row = lambda shape: pl.BlockSpec(shape, lambda i: (i, 0))
    return pl.pallas_call(
        _tc_final_body,
        grid=grid,
        in_specs=[row((RB, D_OUT)), row((RB, D_OUT)),
                  row((RB, 16)), row((RB, 16))],
        out_specs=row((RB, D_OUT)),
        out_shape=jax.ShapeDtypeStruct((N, D_OUT), jnp.float32),
    )(a0, a1, c0, c1)


def _pad_idx(src, dst):
    pad = EPAD - E
    fill = jnp.arange(pad, dtype=jnp.int32)
    srcp = jnp.concatenate([src, fill % N]).reshape(EROWS, BLK)
    dstp = jnp.concatenate([dst, N + fill % (NPAD - N)]).reshape(EROWS, BLK)
    return srcp, dstp


@jax.jit
def kernel(feat, edge_index_rel0, edge_index_rel1,
           W1_0, b1_0, W1_1, b1_1, W2_0, b2_0, W2_1, b2_1):
    src0, dst0 = _pad_idx(edge_index_rel0[0], edge_index_rel0[1])
    src1, dst1 = _pad_idx(edge_index_rel1[0], edge_index_rel1[1])
    zrow128 = jnp.zeros((BLK, D_IN), jnp.float32)
    ones_tab = jnp.ones((N, D_IN), jnp.float32)

    a0p, a1p = _sc_agg(D_IN)(feat, feat, src0, dst0, src1, dst1, zrow128)
    c0p, c1p = _sc_agg(D_IN)(ones_tab, ones_tab,
                             src0, dst0, src1, dst1, zrow128)
    a0, a1 = a0p[:N], a1p[:N]
    c0, c1 = c0p[:N, :16], c1p[:N, :16]

    wh = _tc_mid(a0, a1, c0, c1,
                 W1_0, b1_0, W1_1, b1_1, W2_0, b2_0, W2_1, b2_1)

    g0p, g1p = _sc_agg(2 * D_OUT)(wh, wh, src0, dst0, src1, dst1, zrow128)

    return _tc_final(g0p[:N, :D_OUT], g1p[:N, D_OUT:], c0, c1)
